# Initial kernel scaffold; baseline (speedup 1.0000x reference)
import jax
import jax.numpy as jnp
from jax.experimental import pallas as pl


def kernel(x, scale1_cluster_map, scale1_edge_index):
    raise NotImplementedError("write your pallas kernel here")


# speedup vs baseline: 6.9466x; 6.9466x over previous
import jax
import jax.numpy as jnp
from jax import lax
from jax.experimental import pallas as pl
from jax.experimental.pallas import tpu as pltpu
from jax.experimental.pallas import tpu_sc as plsc

NUM_SEGMENTS = 25000

NC = 2
NS = 16
L = 16


def _build(n, m, d, interpret=False):
    dh = d // NC
    CH = 128
    n_full = n // CH
    n_tail = n - n_full * CH
    CB = 40
    assert m % CB == 0 and CB % 8 == 0 and dh % L == 0
    n_out = m // CB
    cb_pad = ((CB + 2 * L - 1) // L) * L
    tail = max(n_tail, 8)

    mesh = plsc.VectorSubcoreMesh(
        core_axis_name="c", subcore_axis_name="s",
        num_cores=NC, num_subcores=NS)

    def body(x_hbm, cm_hbm, out_hbm, acc, cnt,
             idx2, idx_t, feat2, feat_t, ones_b, vbuf, cbuf,
             sld0, sld1, ssc0, ssc1):
        sld = (sld0, sld1)
        ssc = (ssc0, ssc1)
        cid = lax.axis_index("c")
        sid = lax.axis_index("s")
        col0 = cid * dh

        zvec = jnp.zeros((L,), jnp.float32)
        ovec = jnp.ones((L,), jnp.float32)

        def zrow(i, _):
            for j in range(dh // L):
                vbuf[0, i, pl.ds(j * L, L)] = zvec
            return 0
        lax.fori_loop(0, CB, zrow, 0)
        for B in range(2):
            for j in range(cb_pad // L):
                cbuf[B, pl.ds(j * L, L)] = zvec
        for j in range(CH // L):
            ones_b[pl.ds(j * L, L)] = ovec

        def issue_loads(c, B):
            r0 = c * CH
            pltpu.async_copy(cm_hbm.at[pl.ds(r0, CH)], idx2.at[B], sld[B])
            pltpu.async_copy(x_hbm.at[pl.ds(r0, CH), pl.ds(col0, dh)],
                             feat2.at[B], sld[B])

        def wait_loads(c, B):
            r0 = c * CH
            pltpu.make_async_copy(
                cm_hbm.at[pl.ds(r0, CH)], idx2.at[B], sld[B]).wait()
            pltpu.make_async_copy(
                x_hbm.at[pl.ds(r0, CH), pl.ds(col0, dh)],
                feat2.at[B], sld[B]).wait()

        def issue_scats(B):
            pltpu.async_copy(feat2.at[B], acc.at[idx2.at[B]], ssc[B], add=True)
            pltpu.async_copy(ones_b, cnt.at[idx2.at[B]], ssc[B], add=True)

        def wait_scats(B):
            pltpu.make_async_copy(
                feat2.at[B], acc.at[idx2.at[B]], ssc[B]).wait()
            pltpu.make_async_copy(
                ones_b, cnt.at[idx2.at[B]], ssc[B]).wait()

        c0 = sid * n_full // NS
        c1 = (sid + 1) * n_full // NS

        @pl.when(c0 < c1)
        def _():
            issue_loads(c0, 0)

        k0 = sid * n_out // NS
        k1 = (sid + 1) * n_out // NS

        ZW = 6

        def zissue(k):
            r0 = k * CB
            pltpu.async_copy(vbuf.at[0], acc.at[pl.ds(r0, CB)], ssc0)
            pltpu.async_copy(cbuf.at[0, pl.ds(0, CB)], cnt.at[pl.ds(r0, CB)],
                             ssc0)

        def zwait(k):
            r0 = k * CB
            pltpu.make_async_copy(
                vbuf.at[0], acc.at[pl.ds(r0, CB)], ssc0).wait()
            pltpu.make_async_copy(
                cbuf.at[0, pl.ds(0, CB)], cnt.at[pl.ds(r0, CB)], ssc0).wait()

        def zchunk(k, _):
            zissue(k)

            @pl.when(k - ZW >= k0)
            def _():
                zwait(k - ZW)
            return 0

        with jax.named_scope("zero_phase"):
            lax.fori_loop(k0, k1, zchunk, 0)

            def zdrain(k, _):
                zwait(k)
                return 0
            lax.fori_loop(jnp.maximum(k0, k1 - ZW), k1, zdrain, 0)
            plsc.subcore_barrier()

        def achunk(c, _):
            def step(B):
                wait_loads(c, B)

                @pl.when(c + 1 < c1)
                def _():
                    @pl.when(c > c0)
                    def _():
                        wait_scats(1 - B)
                    issue_loads(c + 1, 1 - B)
                issue_scats(B)

            b_is0 = ((c - c0) % 2) == 0

            @pl.when(b_is0)
            def _():
                step(0)

            @pl.when(jnp.logical_not(b_is0))
            def _():
                step(1)
            return 0

        with jax.named_scope("accum_phase"):
            lax.fori_loop(c0, c1, achunk, 0)

        nch = c1 - c0

        def drain(c):
            p = (c - c0) % 2

            @pl.when(p == 0)
            def _():
                wait_scats(0)

            @pl.when(p == 1)
            def _():
                wait_scats(1)

        with jax.named_scope("accum_drain"):
            @pl.when(nch >= 2)
            def _():
                drain(c1 - 2)

            @pl.when(nch >= 1)
            def _():
                drain(c1 - 1)

        if n_tail:
            @pl.when(sid == NS - 1)
            def _():
                r0 = n_full * CH
                pltpu.sync_copy(cm_hbm.at[pl.ds(r0, n_tail)],
                                idx_t.at[pl.ds(0, n_tail)])
                pltpu.sync_copy(x_hbm.at[pl.ds(r0, n_tail), pl.ds(col0, dh)],
                                feat_t.at[pl.ds(0, n_tail)])
                pltpu.sync_copy(feat_t.at[pl.ds(0, n_tail)],
                                acc.at[idx_t], add=True)
                pltpu.sync_copy(ones_b.at[pl.ds(0, n_tail)],
                                cnt.at[idx_t], add=True)
        plsc.subcore_barrier()

        def issue_dload(k, B):
            r0 = k * CB
            pltpu.async_copy(acc.at[pl.ds(r0, CB)], vbuf.at[B], sld[B])
            pltpu.async_copy(cnt.at[pl.ds(r0, CB)],
                             cbuf.at[B, pl.ds(0, CB)], sld[B])

        def wait_dload(k, B):
            r0 = k * CB
            pltpu.make_async_copy(
                acc.at[pl.ds(r0, CB)], vbuf.at[B], sld[B]).wait()
            pltpu.make_async_copy(
                cnt.at[pl.ds(r0, CB)], cbuf.at[B, pl.ds(0, CB)], sld[B]).wait()

        def issue_dstore(k, B):
            r0 = k * CB
            pltpu.async_copy(
                vbuf.at[B], out_hbm.at[pl.ds(r0, CB), pl.ds(col0, dh)], ssc[B])

        def wait_dstore(k, B):
            r0 = k * CB
            pltpu.make_async_copy(
                vbuf.at[B],
                out_hbm.at[pl.ds(r0, CB), pl.ds(col0, dh)], ssc[B]).wait()

        @pl.when(k0 < k1)
        def _():
            issue_dload(k0, 0)

        def dchunk(k, _):
            def dstep(B):
                wait_dload(k, B)

                @pl.when(k + 1 < k1)
                def _():
                    @pl.when(k > k0)
                    def _():
                        wait_dstore(k - 1, 1 - B)
                    issue_dload(k + 1, 1 - B)
                for j in range(cb_pad // L):
                    v = cbuf[B, pl.ds(j * L, L)]
                    cbuf[B, pl.ds(j * L, L)] = 1.0 / jnp.maximum(v, 1.0)

                def drow(i, _):
                    r = cbuf[B, pl.ds(i, L)][0]
                    for j in range(dh // L):
                        v = vbuf[B, i, pl.ds(j * L, L)]
                        vbuf[B, i, pl.ds(j * L, L)] = v * r
                    return 0
                lax.fori_loop(0, CB, drow, 0)
                issue_dstore(k, B)

            b_is0 = ((k - k0) % 2) == 0

            @pl.when(b_is0)
            def _():
                dstep(0)

            @pl.when(jnp.logical_not(b_is0))
            def _():
                dstep(1)
            return 0

        with jax.named_scope("divide_phase"):
            lax.fori_loop(k0, k1, dchunk, 0)

        def ddrain(k):
            p = (k - k0) % 2

            @pl.when(p == 0)
            def _():
                wait_dstore(k, 0)

            @pl.when(p == 1)
            def _():
                wait_dstore(k, 1)

        nko = k1 - k0

        @pl.when(nko >= 2)
        def _():
            ddrain(k1 - 2)

        @pl.when(nko >= 1)
        def _():
            ddrain(k1 - 1)

    return pl.kernel(
        body,
        out_type=jax.ShapeDtypeStruct((m, d), jnp.float32),
        mesh=mesh,
        interpret=interpret,
        compiler_params=pltpu.CompilerParams(use_tc_tiling_on_sc=False),
        scratch_types=[
            pltpu.VMEM_SHARED((m, dh), jnp.float32),
            pltpu.VMEM_SHARED((m,), jnp.float32),
            pltpu.VMEM((2, CH), jnp.int32),
            pltpu.VMEM((tail,), jnp.int32),
            pltpu.VMEM((2, CH, dh), jnp.float32),
            pltpu.VMEM((tail, dh), jnp.float32),
            pltpu.VMEM((CH,), jnp.float32),
            pltpu.VMEM((2, CB, dh), jnp.float32),
            pltpu.VMEM((2, cb_pad), jnp.float32),
            pltpu.SemaphoreType.DMA,
            pltpu.SemaphoreType.DMA,
            pltpu.SemaphoreType.DMA,
            pltpu.SemaphoreType.DMA,
        ],
    )


def kernel(x, scale1_cluster_map, scale1_edge_index):
    n, d = x.shape
    cm = scale1_cluster_map.astype(jnp.int32)
    pooled = _build(n, NUM_SEGMENTS, d)(x, cm)
    return pooled, scale1_edge_index
